# baseline (speedup 1.0000x reference)
import functools

import numpy as np

import jax
import jax.numpy as jnp
from jax import lax
from jax.experimental import pallas as pl
from jax.experimental.pallas import tpu as pltpu
from jax.experimental.pallas import tpu_sc as plsc

_NC = 2
_NS = 16
_L = 16

_BIGI = 2**30
_INF = float("inf")


def _match_body(nq, nqp, ng, gtq, mt,
                pred_hbm, anch_hbm, gt_hbm, out_hbm, slab, coords, gtv, resbuf):
    wid = lax.axis_index("s") * _NC + lax.axis_index("c")
    q = wid % 4
    src = (wid // 4) % 2
    b = wid // 8
    ng_local = jnp.minimum(gtq, ng - q * gtq)

    @pl.when(src == 0)
    def _():
        pltpu.sync_copy(pred_hbm.at[b], slab)

    @pl.when(src == 1)
    def _():
        pltpu.sync_copy(anch_hbm.at[b], slab)
    pltpu.sync_copy(gt_hbm.at[b], gtv)

    lane = lax.broadcasted_iota(jnp.int32, (_L,), 0)

    def transform(c, _):
        base = c * (4 * _L)
        for u in range(4):
            qvec = base + u * _L + lane
            idx = jnp.minimum(qvec, nq - 1) * 4
            msk = qvec < nq
            x0 = plsc.load_gather(slab, [idx])
            y0 = plsc.load_gather(slab, [idx + 1])
            x1 = plsc.load_gather(slab, [idx + 2])
            y1 = plsc.load_gather(slab, [idx + 3])
            s = pl.ds(c * (4 * _L) + u * _L, _L)
            big = jnp.full((_L,), 1e9, jnp.float32)
            coords[0, s] = jnp.where(msk, (x0 + x1) * 0.5, big)
            coords[1, s] = jnp.where(msk, (y0 + y1) * 0.5, big)
            coords[2, s] = jnp.where(msk, x1 - x0, big)
            coords[3, s] = jnp.where(msk, y1 - y0, big)
        return 0

    lax.fori_loop(0, nqp // (4 * _L), transform, 0)

    def per_gt(gl, outcarry):
        gg = q * gtq + gl
        gx0 = plsc.load_gather(gtv, [jnp.full((_L,), gg, jnp.int32),
                                     jnp.full((_L,), 0, jnp.int32)])
        gy0 = plsc.load_gather(gtv, [jnp.full((_L,), gg, jnp.int32),
                                     jnp.full((_L,), 1, jnp.int32)])
        gx1 = plsc.load_gather(gtv, [jnp.full((_L,), gg, jnp.int32),
                                     jnp.full((_L,), 2, jnp.int32)])
        gy1 = plsc.load_gather(gtv, [jnp.full((_L,), gg, jnp.int32),
                                     jnp.full((_L,), 3, jnp.int32)])
        gcx = (gx0 + gx1) * 0.5
        gcy = (gy0 + gy1) * 0.5
        gw = gx1 - gx0
        gh = gy1 - gy0

        def chunk(c, st):
            m1, m2, m3, m4, i1, i2, i3, i4, iv = st
            vs = []
            for u in range(4):
                s = pl.ds(c * (4 * _L) + u * _L, _L)
                d0 = jnp.abs(coords[0, s] - gcx)
                d1 = jnp.abs(coords[1, s] - gcy)
                d2 = jnp.abs(coords[2, s] - gw)
                d3 = jnp.abs(coords[3, s] - gh)
                vs.append((d0 + d2) + (d1 + d3))
            for u in range(4):
                v = vs[u]
                ivu = iv + u * _L
                c1 = v < m1
                c2 = v < m2
                c3 = v < m3
                c4 = v < m4
                m4 = jnp.where(c4, jnp.where(c3, m3, v), m4)
                i4 = jnp.where(c4, jnp.where(c3, i3, ivu), i4)
                m3 = jnp.where(c3, jnp.where(c2, m2, v), m3)
                i3 = jnp.where(c3, jnp.where(c2, i2, ivu), i3)
                m2 = jnp.where(c2, jnp.where(c1, m1, v), m2)
                i2 = jnp.where(c2, jnp.where(c1, i1, ivu), i2)
                m1 = jnp.where(c1, v, m1)
                i1 = jnp.where(c1, ivu, i1)
            return (m1, m2, m3, m4, i1, i2, i3, i4, iv + 4 * _L)

        st0 = (
            jnp.full((_L,), _INF, jnp.float32), jnp.full((_L,), _INF, jnp.float32),
            jnp.full((_L,), _INF, jnp.float32), jnp.full((_L,), _INF, jnp.float32),
            jnp.full((_L,), _BIGI, jnp.int32), jnp.full((_L,), _BIGI, jnp.int32),
            jnp.full((_L,), _BIGI, jnp.int32), jnp.full((_L,), _BIGI, jnp.int32),
            lane,
        )
        st = lax.fori_loop(0, nqp // (4 * _L), chunk, st0)
        m = list(st[0:4])
        ii = list(st[4:8])

        outs = list(outcarry)
        glmask = lane == gl
        for p in range(mt):
            smin = jnp.min(m[0])
            sidx = jnp.min(jnp.where(m[0] == smin, ii[0], _BIGI))
            outs[p] = jnp.where(glmask, sidx, outs[p])
            chosen = (m[0] == smin) & (ii[0] == sidx)
            for j in range(mt - 1):
                m[j] = jnp.where(chosen, m[j + 1], m[j])
                ii[j] = jnp.where(chosen, ii[j + 1], ii[j])
            m[mt - 1] = jnp.where(chosen, _INF, m[mt - 1])
            ii[mt - 1] = jnp.where(chosen, _BIGI, ii[mt - 1])
        return tuple(outs)

    outs0 = tuple(jnp.zeros((_L,), jnp.int32) for _ in range(mt))
    outs = lax.fori_loop(0, ng_local, per_gt, outs0)
    for k in range(mt):
        resbuf[k, :] = outs[k]
    pltpu.sync_copy(resbuf, out_hbm.at[wid])


def kernel(pred_boxes, anchors, gt_boxes, match_times):
    mt_arr = jnp.asarray(match_times)
    mt = mt_arr.shape[0] if mt_arr.ndim == 1 else int(match_times)
    bs, nq = pred_boxes.shape[:2]
    ng = gt_boxes.shape[1]
    nqp = ((nq + 4 * _L - 1) // (4 * _L)) * (4 * _L)
    gtq = (ng + 3) // 4

    pred_flat = pred_boxes.reshape(bs, nq * 4)
    anch_flat = anchors.reshape(bs, nq * 4)

    nw = _NC * _NS
    mesh = plsc.VectorSubcoreMesh(core_axis_name="c", subcore_axis_name="s")
    body = functools.partial(_match_body, nq, nqp, ng, gtq, mt)
    run = pl.kernel(
        body,
        out_type=jax.ShapeDtypeStruct((nw, mt, _L), jnp.int32),
        mesh=mesh,
        scratch_types=[
            pltpu.VMEM((nq * 4,), jnp.float32),
            pltpu.VMEM((4, nqp), jnp.float32),
            pltpu.VMEM((ng, 4), jnp.float32),
            pltpu.VMEM((mt, _L), jnp.int32),
        ],
        compiler_params=pltpu.CompilerParams(needs_layout_passes=False),
    )
    out = run(pred_flat, anch_flat, gt_boxes)

    bidx, jidx, gidx = np.meshgrid(np.arange(bs), np.arange(mt), np.arange(ng),
                                   indexing="ij")
    sq, sl = np.divmod(gidx, gtq)
    flat = lambda s: (((bidx * 8 + s * 4 + sq) * mt + jidx) * _L + sl)
    perm = np.concatenate([flat(0), flat(1)], axis=-1).reshape(bs, mt * 2 * ng)
    idx_i = out.reshape(-1)[jnp.asarray(perm)]

    row_j = jnp.tile(jnp.concatenate([jnp.arange(ng), jnp.arange(ng)]), mt)
    idx_j = jnp.broadcast_to(row_j, (bs, mt * 2 * ng))
    return idx_i, idx_j


# speedup vs baseline: 1.0599x; 1.0599x over previous
import functools

import numpy as np

import jax
import jax.numpy as jnp
from jax import lax
from jax.experimental import pallas as pl
from jax.experimental.pallas import tpu as pltpu
from jax.experimental.pallas import tpu_sc as plsc

_NC = 2
_NS = 16
_L = 16

_BIGI = 2**30
_INF = float("inf")


def _match_body(nq, nqp, ng, gtq, mt,
                pred_hbm, anch_hbm, gt_hbm, out_hbm, slab, coords, gtv, resbuf):
    wid = lax.axis_index("s") * _NC + lax.axis_index("c")
    q = wid % 4
    src = (wid // 4) % 2
    b = wid // 8
    ng_local = jnp.minimum(gtq, ng - q * gtq)

    @pl.when(src == 0)
    def _():
        pltpu.sync_copy(pred_hbm.at[b], slab)

    @pl.when(src == 1)
    def _():
        pltpu.sync_copy(anch_hbm.at[b], slab)
    pltpu.sync_copy(gt_hbm.at[b], gtv)

    lane = lax.broadcasted_iota(jnp.int32, (_L,), 0)

    def transform(c, _):
        base = c * (4 * _L)
        for u in range(4):
            qvec = base + u * _L + lane
            idx = jnp.minimum(qvec, nq - 1) * 4
            msk = qvec < nq
            x0 = plsc.load_gather(slab, [idx])
            y0 = plsc.load_gather(slab, [idx + 1])
            x1 = plsc.load_gather(slab, [idx + 2])
            y1 = plsc.load_gather(slab, [idx + 3])
            s = pl.ds(c * (4 * _L) + u * _L, _L)
            big = jnp.full((_L,), 1e9, jnp.float32)
            coords[0, s] = jnp.where(msk, (x0 + x1) * 0.5, big)
            coords[1, s] = jnp.where(msk, (y0 + y1) * 0.5, big)
            coords[2, s] = jnp.where(msk, x1 - x0, big)
            coords[3, s] = jnp.where(msk, y1 - y0, big)
        return 0

    lax.fori_loop(0, nqp // (4 * _L), transform, 0)

    def per_gt(gl, outcarry):
        gg = q * gtq + gl
        gx0 = plsc.load_gather(gtv, [jnp.full((_L,), gg, jnp.int32),
                                     jnp.full((_L,), 0, jnp.int32)])
        gy0 = plsc.load_gather(gtv, [jnp.full((_L,), gg, jnp.int32),
                                     jnp.full((_L,), 1, jnp.int32)])
        gx1 = plsc.load_gather(gtv, [jnp.full((_L,), gg, jnp.int32),
                                     jnp.full((_L,), 2, jnp.int32)])
        gy1 = plsc.load_gather(gtv, [jnp.full((_L,), gg, jnp.int32),
                                     jnp.full((_L,), 3, jnp.int32)])
        gcx = (gx0 + gx1) * 0.5
        gcy = (gy0 + gy1) * 0.5
        gw = gx1 - gx0
        gh = gy1 - gy0

        def chunk(c, st):
            m1, m2, m3, m4, i1, i2, i3, i4, iv = st
            vs = []
            for u in range(4):
                s = pl.ds(c * (4 * _L) + u * _L, _L)
                d0 = jnp.abs(coords[0, s] - gcx)
                d1 = jnp.abs(coords[1, s] - gcy)
                d2 = jnp.abs(coords[2, s] - gw)
                d3 = jnp.abs(coords[3, s] - gh)
                vs.append((d0 + d2) + (d1 + d3))
            for u in range(4):
                v = vs[u]
                ivu = iv + u * _L
                c1 = v < m1
                c2 = v < m2
                c3 = v < m3
                c4 = v < m4
                m4 = jnp.where(c4, jnp.where(c3, m3, v), m4)
                i4 = jnp.where(c4, jnp.where(c3, i3, ivu), i4)
                m3 = jnp.where(c3, jnp.where(c2, m2, v), m3)
                i3 = jnp.where(c3, jnp.where(c2, i2, ivu), i3)
                m2 = jnp.where(c2, jnp.where(c1, m1, v), m2)
                i2 = jnp.where(c2, jnp.where(c1, i1, ivu), i2)
                m1 = jnp.where(c1, v, m1)
                i1 = jnp.where(c1, ivu, i1)
            return (m1, m2, m3, m4, i1, i2, i3, i4, iv + 4 * _L)

        st0 = (
            jnp.full((_L,), _INF, jnp.float32), jnp.full((_L,), _INF, jnp.float32),
            jnp.full((_L,), _INF, jnp.float32), jnp.full((_L,), _INF, jnp.float32),
            jnp.full((_L,), _BIGI, jnp.int32), jnp.full((_L,), _BIGI, jnp.int32),
            jnp.full((_L,), _BIGI, jnp.int32), jnp.full((_L,), _BIGI, jnp.int32),
            lane,
        )
        st = lax.fori_loop(0, nqp // (4 * _L), chunk, st0)
        m = list(st[0:4])
        ii = list(st[4:8])

        outs = list(outcarry)
        glmask = lane == gl
        for p in range(mt):
            smin = jnp.min(m[0])
            sidx = jnp.min(jnp.where(m[0] == smin, ii[0], _BIGI))
            outs[p] = jnp.where(glmask, sidx, outs[p])
            chosen = (m[0] == smin) & (ii[0] == sidx)
            for j in range(mt - 1):
                m[j] = jnp.where(chosen, m[j + 1], m[j])
                ii[j] = jnp.where(chosen, ii[j + 1], ii[j])
            m[mt - 1] = jnp.where(chosen, _INF, m[mt - 1])
            ii[mt - 1] = jnp.where(chosen, _BIGI, ii[mt - 1])
        return tuple(outs)

    outs0 = tuple(jnp.zeros((_L,), jnp.int32) for _ in range(mt))
    outs = lax.fori_loop(0, ng_local, per_gt, outs0)
    for k in range(mt):
        resbuf[k, :] = outs[k]
    pltpu.sync_copy(resbuf, out_hbm.at[wid])


def kernel(pred_boxes, anchors, gt_boxes, match_times):
    mt_arr = jnp.asarray(match_times)
    mt = mt_arr.shape[0] if mt_arr.ndim == 1 else int(match_times)
    bs, nq = pred_boxes.shape[:2]
    ng = gt_boxes.shape[1]
    nqp = ((nq + 4 * _L - 1) // (4 * _L)) * (4 * _L)
    gtq = (ng + 3) // 4

    pred_flat = pred_boxes.reshape(bs, nq * 4)
    anch_flat = anchors.reshape(bs, nq * 4)

    nw = _NC * _NS
    mesh = plsc.VectorSubcoreMesh(core_axis_name="c", subcore_axis_name="s")
    body = functools.partial(_match_body, nq, nqp, ng, gtq, mt)
    run = pl.kernel(
        body,
        out_type=jax.ShapeDtypeStruct((nw, mt, _L), jnp.int32),
        mesh=mesh,
        scratch_types=[
            pltpu.VMEM((nq * 4,), jnp.float32),
            pltpu.VMEM((4, nqp), jnp.float32),
            pltpu.VMEM((ng, 4), jnp.float32),
            pltpu.VMEM((mt, _L), jnp.int32),
        ],
        compiler_params=pltpu.CompilerParams(needs_layout_passes=False),
    )
    out = run(pred_flat, anch_flat, gt_boxes)

    res = out.reshape(bs, 2, 4, mt, _L)
    parts = [res[:, :, qq, :, :min(gtq, ng - qq * gtq)] for qq in range(4)]
    gcat = jnp.concatenate(parts, axis=-1)
    idx_i = jnp.transpose(gcat, (0, 2, 1, 3)).reshape(bs, mt * 2 * ng)

    row_j = jnp.tile(jnp.concatenate([jnp.arange(ng), jnp.arange(ng)]), mt)
    idx_j = jnp.broadcast_to(row_j, (bs, mt * 2 * ng))
    return idx_i, idx_j


# speedup vs baseline: 1.0644x; 1.0043x over previous
import functools

import numpy as np

import jax
import jax.numpy as jnp
from jax import lax
from jax.experimental import pallas as pl
from jax.experimental.pallas import tpu as pltpu
from jax.experimental.pallas import tpu_sc as plsc

_NC = 2
_NS = 16
_L = 16

_BIGI = 2**30
_INF = float("inf")


def _match_body(nq, nqp, ng, gtq, mt,
                boxes_hbm, gt_hbm, out_hbm, slab, coords, gtv, resbuf):
    wid = lax.axis_index("s") * _NC + lax.axis_index("c")
    q = wid % 4
    src = (wid // 4) % 2
    b = wid // 8
    ng_local = jnp.minimum(gtq, ng - q * gtq)

    pltpu.sync_copy(boxes_hbm.at[src, b], slab)
    pltpu.sync_copy(gt_hbm.at[b], gtv)

    lane = lax.broadcasted_iota(jnp.int32, (_L,), 0)

    def transform(c, _):
        base = c * (4 * _L)
        for u in range(4):
            qvec = base + u * _L + lane
            idx = jnp.minimum(qvec, nq - 1) * 4
            msk = qvec < nq
            x0 = plsc.load_gather(slab, [idx])
            y0 = plsc.load_gather(slab, [idx + 1])
            x1 = plsc.load_gather(slab, [idx + 2])
            y1 = plsc.load_gather(slab, [idx + 3])
            s = pl.ds(c * (4 * _L) + u * _L, _L)
            big = jnp.full((_L,), 1e9, jnp.float32)
            coords[0, s] = jnp.where(msk, (x0 + x1) * 0.5, big)
            coords[1, s] = jnp.where(msk, (y0 + y1) * 0.5, big)
            coords[2, s] = jnp.where(msk, x1 - x0, big)
            coords[3, s] = jnp.where(msk, y1 - y0, big)
        return 0

    lax.fori_loop(0, nqp // (4 * _L), transform, 0)

    def per_gt(gl, outcarry):
        gg = q * gtq + gl
        gx0 = plsc.load_gather(gtv, [jnp.full((_L,), gg, jnp.int32),
                                     jnp.full((_L,), 0, jnp.int32)])
        gy0 = plsc.load_gather(gtv, [jnp.full((_L,), gg, jnp.int32),
                                     jnp.full((_L,), 1, jnp.int32)])
        gx1 = plsc.load_gather(gtv, [jnp.full((_L,), gg, jnp.int32),
                                     jnp.full((_L,), 2, jnp.int32)])
        gy1 = plsc.load_gather(gtv, [jnp.full((_L,), gg, jnp.int32),
                                     jnp.full((_L,), 3, jnp.int32)])
        gcx = (gx0 + gx1) * 0.5
        gcy = (gy0 + gy1) * 0.5
        gw = gx1 - gx0
        gh = gy1 - gy0

        def chunk(c, st):
            m1, m2, m3, m4, i1, i2, i3, i4, iv = st
            vs = []
            for u in range(4):
                s = pl.ds(c * (4 * _L) + u * _L, _L)
                d0 = jnp.abs(coords[0, s] - gcx)
                d1 = jnp.abs(coords[1, s] - gcy)
                d2 = jnp.abs(coords[2, s] - gw)
                d3 = jnp.abs(coords[3, s] - gh)
                vs.append((d0 + d2) + (d1 + d3))
            for u in range(4):
                v = vs[u]
                ivu = iv + u * _L
                c1 = v < m1
                c2 = v < m2
                c3 = v < m3
                c4 = v < m4
                m4 = jnp.where(c4, jnp.where(c3, m3, v), m4)
                i4 = jnp.where(c4, jnp.where(c3, i3, ivu), i4)
                m3 = jnp.where(c3, jnp.where(c2, m2, v), m3)
                i3 = jnp.where(c3, jnp.where(c2, i2, ivu), i3)
                m2 = jnp.where(c2, jnp.where(c1, m1, v), m2)
                i2 = jnp.where(c2, jnp.where(c1, i1, ivu), i2)
                m1 = jnp.where(c1, v, m1)
                i1 = jnp.where(c1, ivu, i1)
            return (m1, m2, m3, m4, i1, i2, i3, i4, iv + 4 * _L)

        st0 = (
            jnp.full((_L,), _INF, jnp.float32), jnp.full((_L,), _INF, jnp.float32),
            jnp.full((_L,), _INF, jnp.float32), jnp.full((_L,), _INF, jnp.float32),
            jnp.full((_L,), _BIGI, jnp.int32), jnp.full((_L,), _BIGI, jnp.int32),
            jnp.full((_L,), _BIGI, jnp.int32), jnp.full((_L,), _BIGI, jnp.int32),
            lane,
        )
        st = lax.fori_loop(0, nqp // (4 * _L), chunk, st0)
        m = list(st[0:4])
        ii = list(st[4:8])

        outs = list(outcarry)
        glmask = lane == gl
        for p in range(mt):
            smin = jnp.min(m[0])
            sidx = jnp.min(jnp.where(m[0] == smin, ii[0], _BIGI))
            outs[p] = jnp.where(glmask, sidx, outs[p])
            chosen = (m[0] == smin) & (ii[0] == sidx)
            for j in range(mt - 1):
                m[j] = jnp.where(chosen, m[j + 1], m[j])
                ii[j] = jnp.where(chosen, ii[j + 1], ii[j])
            m[mt - 1] = jnp.where(chosen, _INF, m[mt - 1])
            ii[mt - 1] = jnp.where(chosen, _BIGI, ii[mt - 1])
        return tuple(outs)

    outs0 = tuple(jnp.zeros((_L,), jnp.int32) for _ in range(mt))
    outs = lax.fori_loop(0, ng_local, per_gt, outs0)
    for k in range(mt):
        resbuf[k, :] = outs[k]
    pltpu.sync_copy(resbuf, out_hbm.at[wid])


def kernel(pred_boxes, anchors, gt_boxes, match_times):
    mt_arr = jnp.asarray(match_times)
    mt = mt_arr.shape[0] if mt_arr.ndim == 1 else int(match_times)
    bs, nq = pred_boxes.shape[:2]
    ng = gt_boxes.shape[1]
    nqp = ((nq + 4 * _L - 1) // (4 * _L)) * (4 * _L)
    gtq = (ng + 3) // 4

    boxes_flat = jnp.stack([pred_boxes.reshape(bs, nq * 4),
                            anchors.reshape(bs, nq * 4)])

    nw = _NC * _NS
    mesh = plsc.VectorSubcoreMesh(core_axis_name="c", subcore_axis_name="s")
    body = functools.partial(_match_body, nq, nqp, ng, gtq, mt)
    run = pl.kernel(
        body,
        out_type=jax.ShapeDtypeStruct((nw, mt, _L), jnp.int32),
        mesh=mesh,
        scratch_types=[
            pltpu.VMEM((nq * 4,), jnp.float32),
            pltpu.VMEM((4, nqp), jnp.float32),
            pltpu.VMEM((ng, 4), jnp.float32),
            pltpu.VMEM((mt, _L), jnp.int32),
        ],
        compiler_params=pltpu.CompilerParams(needs_layout_passes=False),
    )
    out = run(boxes_flat, gt_boxes)

    res = out.reshape(bs, 2, 4, mt, _L)
    parts = [res[:, :, qq, :, :min(gtq, ng - qq * gtq)] for qq in range(4)]
    gcat = jnp.concatenate(parts, axis=-1)
    idx_i = jnp.transpose(gcat, (0, 2, 1, 3)).reshape(bs, mt * 2 * ng)

    row_j = jnp.tile(jnp.concatenate([jnp.arange(ng), jnp.arange(ng)]), mt)
    idx_j = jnp.broadcast_to(row_j, (bs, mt * 2 * ng))
    return idx_i, idx_j


# speedup vs baseline: 1.4644x; 1.3758x over previous
import functools

import jax
import jax.numpy as jnp
from jax import lax
from jax.experimental import pallas as pl
from jax.experimental.pallas import tpu as pltpu
from jax.experimental.pallas import tpu_sc as plsc

_NC = 2
_NS = 16
_L = 16

_BIGI = 2**30
_INF = float("inf")


def _match_body(nq, nqp, ng, gtq, mt, boxes_hbm, gt_hbm, out_hbm, coords, gtv, resbuf):
    wid = lax.axis_index("s") * _NC + lax.axis_index("c")
    q = wid % 4
    src = (wid // 4) % 2
    b = wid // 8
    ng_local = jnp.minimum(gtq, ng - q * gtq)

    pltpu.sync_copy(boxes_hbm.at[src, b], coords)
    pltpu.sync_copy(gt_hbm.at[b], gtv)

    def transform(c, _):
        for u in range(4):
            s = pl.ds(c * (4 * _L) + u * _L, _L)
            x0 = coords[0, s]
            y0 = coords[1, s]
            x1 = coords[2, s]
            y1 = coords[3, s]
            coords[0, s] = (x0 + x1) * 0.5
            coords[1, s] = (y0 + y1) * 0.5
            coords[2, s] = x1 - x0
            coords[3, s] = y1 - y0
        return 0

    lax.fori_loop(0, nqp // (4 * _L), transform, 0)

    lane = lax.broadcasted_iota(jnp.int32, (_L,), 0)

    def per_gt(gl, outcarry):
        gg = q * gtq + gl
        gx0 = gtv[gg, 0, :]
        gy0 = gtv[gg, 1, :]
        gx1 = gtv[gg, 2, :]
        gy1 = gtv[gg, 3, :]
        gcx = (gx0 + gx1) * 0.5
        gcy = (gy0 + gy1) * 0.5
        gw = gx1 - gx0
        gh = gy1 - gy0

        def chunk(c, st):
            m1, m2, m3, m4, i1, i2, i3, i4, iv = st
            vs = []
            for u in range(4):
                s = pl.ds(c * (4 * _L) + u * _L, _L)
                d0 = jnp.abs(coords[0, s] - gcx)
                d1 = jnp.abs(coords[1, s] - gcy)
                d2 = jnp.abs(coords[2, s] - gw)
                d3 = jnp.abs(coords[3, s] - gh)
                vs.append((d0 + d2) + (d1 + d3))
            for u in range(4):
                v = vs[u]
                ivu = iv + u * _L
                c1 = v < m1
                c2 = v < m2
                c3 = v < m3
                c4 = v < m4
                m4 = jnp.where(c4, jnp.where(c3, m3, v), m4)
                i4 = jnp.where(c4, jnp.where(c3, i3, ivu), i4)
                m3 = jnp.where(c3, jnp.where(c2, m2, v), m3)
                i3 = jnp.where(c3, jnp.where(c2, i2, ivu), i3)
                m2 = jnp.where(c2, jnp.where(c1, m1, v), m2)
                i2 = jnp.where(c2, jnp.where(c1, i1, ivu), i2)
                m1 = jnp.where(c1, v, m1)
                i1 = jnp.where(c1, ivu, i1)
            return (m1, m2, m3, m4, i1, i2, i3, i4, iv + 4 * _L)

        st0 = (
            jnp.full((_L,), _INF, jnp.float32), jnp.full((_L,), _INF, jnp.float32),
            jnp.full((_L,), _INF, jnp.float32), jnp.full((_L,), _INF, jnp.float32),
            jnp.full((_L,), _BIGI, jnp.int32), jnp.full((_L,), _BIGI, jnp.int32),
            jnp.full((_L,), _BIGI, jnp.int32), jnp.full((_L,), _BIGI, jnp.int32),
            lane,
        )
        st = lax.fori_loop(0, nqp // (4 * _L), chunk, st0)
        m = list(st[0:4])
        ii = list(st[4:8])

        outs = list(outcarry)
        glmask = lane == gl
        for p in range(mt):
            smin = jnp.min(m[0])
            sidx = jnp.min(jnp.where(m[0] == smin, ii[0], _BIGI))
            outs[p] = jnp.where(glmask, sidx, outs[p])
            chosen = (m[0] == smin) & (ii[0] == sidx)
            for j in range(mt - 1):
                m[j] = jnp.where(chosen, m[j + 1], m[j])
                ii[j] = jnp.where(chosen, ii[j + 1], ii[j])
            m[mt - 1] = jnp.where(chosen, _INF, m[mt - 1])
            ii[mt - 1] = jnp.where(chosen, _BIGI, ii[mt - 1])
        return tuple(outs)

    outs0 = tuple(jnp.zeros((_L,), jnp.int32) for _ in range(mt))
    outs = lax.fori_loop(0, ng_local, per_gt, outs0)
    for k in range(mt):
        resbuf[k, :] = outs[k]
    pltpu.sync_copy(resbuf, out_hbm.at[wid])


def kernel(pred_boxes, anchors, gt_boxes, match_times):
    mt_arr = jnp.asarray(match_times)
    mt = mt_arr.shape[0] if mt_arr.ndim == 1 else int(match_times)
    bs, nq = pred_boxes.shape[:2]
    ng = gt_boxes.shape[1]
    nqp = ((nq + 4 * _L - 1) // (4 * _L)) * (4 * _L)
    gtq = (ng + 3) // 4

    boxes = jnp.stack([pred_boxes, anchors])
    boxes_t = jnp.transpose(boxes, (0, 1, 3, 2))
    boxes_t = jnp.concatenate(
        [boxes_t, jnp.full((2, bs, 4, nqp - nq), 1e9, jnp.float32)], axis=-1)
    gt_spl = jnp.broadcast_to(gt_boxes.reshape(bs, ng, 4, 1), (bs, ng, 4, _L))

    nw = _NC * _NS
    mesh = plsc.VectorSubcoreMesh(core_axis_name="c", subcore_axis_name="s")
    body = functools.partial(_match_body, nq, nqp, ng, gtq, mt)
    run = pl.kernel(
        body,
        out_type=jax.ShapeDtypeStruct((nw, mt, _L), jnp.int32),
        mesh=mesh,
        scratch_types=[
            pltpu.VMEM((4, nqp), jnp.float32),
            pltpu.VMEM((ng, 4, _L), jnp.float32),
            pltpu.VMEM((mt, _L), jnp.int32),
        ],
        compiler_params=pltpu.CompilerParams(needs_layout_passes=False),
    )
    out = run(boxes_t, gt_spl)

    res = out.reshape(bs, 2, 4, mt, _L)
    parts = [res[:, :, qq, :, :min(gtq, ng - qq * gtq)] for qq in range(4)]
    gcat = jnp.concatenate(parts, axis=-1)
    idx_i = jnp.transpose(gcat, (0, 2, 1, 3)).reshape(bs, mt * 2 * ng)

    row_j = jnp.tile(jnp.concatenate([jnp.arange(ng), jnp.arange(ng)]), mt)
    idx_j = jnp.broadcast_to(row_j, (bs, mt * 2 * ng))
    return idx_i, idx_j
